# Initial kernel scaffold; baseline (speedup 1.0000x reference)
import jax
import jax.numpy as jnp
from jax.experimental import pallas as pl


def kernel(positions, pe):
    raise NotImplementedError("write your pallas kernel here")


# speedup vs baseline: 3.5533x; 3.5533x over previous
import functools
import math

import jax
import jax.numpy as jnp
from jax import lax
from jax.experimental import pallas as pl
from jax.experimental.pallas import tpu as pltpu, tpu_sc as plsc

N = 100000
D = 128
HALF = 64
MAX_LEN = 10000

NC = 2
NS = 16
NW = NC * NS
L = 16

B = 64
GB = 2 * B
NBLK = (N + B - 1) // B + 1
STEPS = (NBLK + NW - 1) // NW
LAST_R0 = N - B

P1_CHUNK = 6256
P1_LAST = N - P1_CHUNK
P1_ITERS = P1_CHUNK // L


def _shuf(v, perm):
    return lax.gather(
        v, perm[:, None],
        dimension_numbers=lax.GatherDimensionNumbers(
            offset_dims=(), collapsed_slice_dims=(0,), start_index_map=(0,)),
        slice_sizes=(1,),
        mode=lax.GatherScatterMode.PROMISE_IN_BOUNDS)


def _sc_body(x_hbm, y_hbm, pe_hbm, out_hbm,
             xbuf, ybuf, pbuf, allbuf, shared, xv, yv, idxx, idxy,
             rowsx, rowsy, sem):
    c = lax.axis_index("c")
    s = lax.axis_index("s")
    wid = s * NC + c

    off = jnp.minimum(s * P1_CHUNK, P1_LAST)
    pltpu.sync_copy(x_hbm.at[pl.ds(off, P1_CHUNK)], xbuf)
    pltpu.sync_copy(y_hbm.at[pl.ds(off, P1_CHUNK)], ybuf)

    def red_body(i, carry):
        mnx, mxx, mny, mxy = carry
        xk = xbuf[pl.ds(i * L, L)]
        yk = ybuf[pl.ds(i * L, L)]
        return (jnp.minimum(mnx, xk), jnp.maximum(mxx, xk),
                jnp.minimum(mny, yk), jnp.maximum(mxy, yk))

    x0 = xbuf[pl.ds(0, L)]
    y0 = ybuf[pl.ds(0, L)]
    mnx, mxx, mny, mxy = lax.fori_loop(
        1, P1_ITERS, red_body, (x0, x0, y0, y0))
    pbuf[0, :] = mnx
    pbuf[1, :] = mxx
    pbuf[2, :] = mny
    pbuf[3, :] = mxy

    pltpu.sync_copy(pbuf, shared.at[s])
    plsc.subcore_barrier()
    pltpu.sync_copy(shared, allbuf)

    mnx, mxx, mny, mxy = allbuf[0, 0, :], allbuf[0, 1, :], allbuf[0, 2, :], allbuf[0, 3, :]
    for t in range(1, NS):
        mnx = jnp.minimum(mnx, allbuf[t, 0, :])
        mxx = jnp.maximum(mxx, allbuf[t, 1, :])
        mny = jnp.minimum(mny, allbuf[t, 2, :])
        mxy = jnp.maximum(mxy, allbuf[t, 3, :])
    lane = lax.iota(jnp.int32, L)
    for sh in (8, 4, 2, 1):
        perm = lax.bitwise_xor(lane, sh)
        mnx = jnp.minimum(mnx, _shuf(mnx, perm))
        mxx = jnp.maximum(mxx, _shuf(mxx, perm))
        mny = jnp.minimum(mny, _shuf(mny, perm))
        mxy = jnp.maximum(mxy, _shuf(mxy, perm))
    mnx_s = mnx
    dx_s = mxx - mnx + 1e-8
    mny_s = mny
    dy_s = mxy - mny + 1e-8

    def blk_body(i, carry):
        b = jnp.minimum(wid + i * NW, NBLK - 1)
        r0 = jnp.minimum(b * B, LAST_R0)
        pltpu.sync_copy(x_hbm.at[pl.ds(r0, B)], xv)
        pltpu.sync_copy(y_hbm.at[pl.ds(r0, B)], yv)
        for k in range(B // L):
            xn = (xv[pl.ds(k * L, L)] - mnx_s) / dx_s
            yn = (yv[pl.ds(k * L, L)] - mny_s) / dy_s
            xi = jnp.clip((xn * float(MAX_LEN)).astype(jnp.int32),
                          0, MAX_LEN - 1)
            yi = jnp.clip((yn * float(MAX_LEN)).astype(jnp.int32),
                          0, MAX_LEN - 1)
            idxx[pl.ds(k * L, L)] = xi
            idxy[pl.ds(k * L, L)] = yi
        cpx = pltpu.async_copy(pe_hbm.at[idxx], rowsx, sem)
        cpy = pltpu.async_copy(pe_hbm.at[idxy], rowsy, sem)
        cpx.wait()
        cpy.wait()
        pltpu.sync_copy(rowsx, out_hbm.at[pl.ds(r0, B), pl.ds(0, HALF)])
        pltpu.sync_copy(rowsy, out_hbm.at[pl.ds(r0, B), pl.ds(HALF, HALF)])
        return carry

    lax.fori_loop(0, STEPS, blk_body, 0)


@functools.partial(jax.jit, static_argnames=())
def _encode(x, y, pe_half):
    mesh = plsc.VectorSubcoreMesh(core_axis_name="c", subcore_axis_name="s")
    run = pl.kernel(
        _sc_body,
        mesh=mesh,
        compiler_params=pltpu.CompilerParams(use_tc_tiling_on_sc=False),
        out_type=jax.ShapeDtypeStruct((N, D), jnp.float32),
        scratch_types=[
            pltpu.VMEM((P1_CHUNK,), jnp.float32),
            pltpu.VMEM((P1_CHUNK,), jnp.float32),
            pltpu.VMEM((4, L), jnp.float32),
            pltpu.VMEM((NS, 4, L), jnp.float32),
            pltpu.VMEM_SHARED((NS, 4, L), jnp.float32),
            pltpu.VMEM((B,), jnp.float32),
            pltpu.VMEM((B,), jnp.float32),
            pltpu.VMEM((B,), jnp.int32),
            pltpu.VMEM((B,), jnp.int32),
            pltpu.VMEM((B, HALF), jnp.float32),
            pltpu.VMEM((B, HALF), jnp.float32),
            pltpu.SemaphoreType.DMA,
        ],
    )
    return run(x, y, pe_half)


def kernel(positions, pe):
    if positions.ndim == 1:
        positions = positions[None, :]
    x = positions[:, 0]
    y = positions[:, 1]
    pe_half = pe[:, :HALF]
    return _encode(x, y, pe_half)


# speedup vs baseline: 5.9050x; 1.6618x over previous
import functools
import math

import jax
import jax.numpy as jnp
from jax import lax
from jax.experimental import pallas as pl
from jax.experimental.pallas import tpu as pltpu, tpu_sc as plsc

N = 100000
D = 128
HALF = 64
MAX_LEN = 10000

NC = 2
NS = 16
NW = NC * NS
L = 16

B = 128
NBLK = (N + B - 1) // B
STEPS = (NBLK + NW - 1) // NW
LAST_R0 = N - B

P1_CHUNK = 6256
P1_LAST = N - P1_CHUNK
P1_ITERS = P1_CHUNK // L


def _shuf(v, perm):
    return lax.gather(
        v, perm[:, None],
        dimension_numbers=lax.GatherDimensionNumbers(
            offset_dims=(), collapsed_slice_dims=(0,), start_index_map=(0,)),
        slice_sizes=(1,),
        mode=lax.GatherScatterMode.PROMISE_IN_BOUNDS)


def _sc_body(x_hbm, y_hbm, pe_hbm, out_hbm,
             xbuf, ybuf, pbuf, allbuf, shared, xv, yv, idxx, idxy,
             rowsx, rowsy, semp, semg, semo):
    c = lax.axis_index("c")
    s = lax.axis_index("s")
    wid = s * NC + c

    off = jnp.minimum(s * P1_CHUNK, P1_LAST)
    pltpu.sync_copy(x_hbm.at[pl.ds(off, P1_CHUNK)], xbuf)
    pltpu.sync_copy(y_hbm.at[pl.ds(off, P1_CHUNK)], ybuf)

    def red_body(i, carry):
        mnx, mxx, mny, mxy = carry
        xk = xbuf[pl.ds(i * L, L)]
        yk = ybuf[pl.ds(i * L, L)]
        return (jnp.minimum(mnx, xk), jnp.maximum(mxx, xk),
                jnp.minimum(mny, yk), jnp.maximum(mxy, yk))

    x0 = xbuf[pl.ds(0, L)]
    y0 = ybuf[pl.ds(0, L)]
    mnx, mxx, mny, mxy = lax.fori_loop(
        1, P1_ITERS, red_body, (x0, x0, y0, y0))
    pbuf[0, :] = mnx
    pbuf[1, :] = mxx
    pbuf[2, :] = mny
    pbuf[3, :] = mxy

    pltpu.sync_copy(pbuf, shared.at[s])
    plsc.subcore_barrier()
    pltpu.sync_copy(shared, allbuf)

    mnx, mxx, mny, mxy = allbuf[0, 0, :], allbuf[0, 1, :], allbuf[0, 2, :], allbuf[0, 3, :]
    for t in range(1, NS):
        mnx = jnp.minimum(mnx, allbuf[t, 0, :])
        mxx = jnp.maximum(mxx, allbuf[t, 1, :])
        mny = jnp.minimum(mny, allbuf[t, 2, :])
        mxy = jnp.maximum(mxy, allbuf[t, 3, :])
    lane = lax.iota(jnp.int32, L)
    for sh in (8, 4, 2, 1):
        perm = lax.bitwise_xor(lane, sh)
        mnx = jnp.minimum(mnx, _shuf(mnx, perm))
        mxx = jnp.maximum(mxx, _shuf(mxx, perm))
        mny = jnp.minimum(mny, _shuf(mny, perm))
        mxy = jnp.maximum(mxy, _shuf(mxy, perm))
    mnx_s = mnx
    dx_s = mxx - mnx + 1e-8
    mny_s = mny
    dy_s = mxy - mny + 1e-8

    def _row0(i):
        b = jnp.minimum(wid + i * NW, NBLK - 1)
        return jnp.minimum(b * B, LAST_R0)

    def _quantize(p):
        for k in range(B // L):
            xn = (xv[p][pl.ds(k * L, L)] - mnx_s) / dx_s
            yn = (yv[p][pl.ds(k * L, L)] - mny_s) / dy_s
            idxx[p][pl.ds(k * L, L)] = jnp.clip(
                (xn * float(MAX_LEN)).astype(jnp.int32), 0, MAX_LEN - 1)
            idxy[p][pl.ds(k * L, L)] = jnp.clip(
                (yn * float(MAX_LEN)).astype(jnp.int32), 0, MAX_LEN - 1)

    def _gather(p):
        cpx = pltpu.async_copy(pe_hbm.at[idxx[p]], rowsx[p], semg[p])
        cpy = pltpu.async_copy(pe_hbm.at[idxy[p]], rowsy[p], semg[p])
        cpx.wait()
        cpy.wait()

    def _issue_writeout(p, r0):
        pltpu.async_copy(rowsx[p],
                         out_hbm.at[pl.ds(r0, B), pl.ds(0, HALF)], semo[p])
        pltpu.async_copy(rowsy[p],
                         out_hbm.at[pl.ds(r0, B), pl.ds(HALF, HALF)], semo[p])

    def _drain_writeout(p):
        pltpu.make_async_copy(
            rowsx[p], out_hbm.at[pl.ds(0, B), pl.ds(0, HALF)], semo[p]).wait()
        pltpu.make_async_copy(
            rowsy[p], out_hbm.at[pl.ds(0, B), pl.ds(HALF, HALF)],
            semo[p]).wait()

    def _issue_pos(p, r0):
        pltpu.async_copy(x_hbm.at[pl.ds(r0, B)], xv[p], semp[p])
        pltpu.async_copy(y_hbm.at[pl.ds(r0, B)], yv[p], semp[p])

    def _drain_pos(p):
        pltpu.make_async_copy(x_hbm.at[pl.ds(0, B)], xv[p], semp[p]).wait()
        pltpu.make_async_copy(y_hbm.at[pl.ds(0, B)], yv[p], semp[p]).wait()

    for i in (0, 1):
        r0 = _row0(i)
        pltpu.sync_copy(x_hbm.at[pl.ds(r0, B)], xv[i])
        pltpu.sync_copy(y_hbm.at[pl.ds(r0, B)], yv[i])
        _quantize(i)
        _gather(i)
        _issue_writeout(i, r0)
    _issue_pos(0, _row0(2))
    _issue_pos(1, _row0(3))

    def phase(i, p):
        r0 = _row0(i)
        _drain_pos(p)
        _quantize(p)
        _issue_pos(p, _row0(i + 2))
        _drain_writeout(p)
        _gather(p)
        _issue_writeout(p, r0)

    def pipe_body(j, carry):
        phase(2 + 2 * j, 0)
        phase(3 + 2 * j, 1)
        return carry

    NPAIR = (STEPS - 2) // 2 + 1
    lax.fori_loop(0, NPAIR, pipe_body, 0)
    for p in (0, 1):
        _drain_pos(p)
        _drain_writeout(p)


@functools.partial(jax.jit, static_argnames=())
def _encode(x, y, pe_half):
    mesh = plsc.VectorSubcoreMesh(core_axis_name="c", subcore_axis_name="s")
    run = pl.kernel(
        _sc_body,
        mesh=mesh,
        compiler_params=pltpu.CompilerParams(use_tc_tiling_on_sc=False),
        out_type=jax.ShapeDtypeStruct((N, D), jnp.float32),
        scratch_types=[
            pltpu.VMEM((P1_CHUNK,), jnp.float32),
            pltpu.VMEM((P1_CHUNK,), jnp.float32),
            pltpu.VMEM((4, L), jnp.float32),
            pltpu.VMEM((NS, 4, L), jnp.float32),
            pltpu.VMEM_SHARED((NS, 4, L), jnp.float32),
            [pltpu.VMEM((B,), jnp.float32)] * 2,
            [pltpu.VMEM((B,), jnp.float32)] * 2,
            [pltpu.VMEM((B,), jnp.int32)] * 2,
            [pltpu.VMEM((B,), jnp.int32)] * 2,
            [pltpu.VMEM((B, HALF), jnp.float32)] * 2,
            [pltpu.VMEM((B, HALF), jnp.float32)] * 2,
            [pltpu.SemaphoreType.DMA] * 2,
            [pltpu.SemaphoreType.DMA] * 2,
            [pltpu.SemaphoreType.DMA] * 2,
        ],
    )
    return run(x, y, pe_half)


def kernel(positions, pe):
    if positions.ndim == 1:
        positions = positions[None, :]
    x = positions[:, 0]
    y = positions[:, 1]
    pe_half = pe[:, :HALF]
    return _encode(x, y, pe_half)
